# baseline (speedup 1.0000x reference)
import jax
import jax.numpy as jnp
from jax.experimental import pallas as pl
from jax.experimental.pallas import tpu as pltpu


_ROWS = 16384
_DIM = 128
_NCHUNK = 8
_CHUNK = _ROWS // _NCHUNK


def _copy_kernel(q_hbm, out_hbm, mask_ref, b0, b1, b2, b3, b4, b5, b6, b7,
                 in_sem, out_sem):
    bufs = (b0, b1, b2, b3, b4, b5, b6, b7)
    for i in range(_NCHUNK):
        pltpu.make_async_copy(
            q_hbm.at[pl.ds(i * _CHUNK, _CHUNK)], bufs[i], in_sem.at[i]
        ).start()
    mask_ref[...] = jnp.ones(mask_ref.shape, dtype=jnp.bool_)
    for i in range(_NCHUNK):
        pltpu.make_async_copy(
            q_hbm.at[pl.ds(i * _CHUNK, _CHUNK)], bufs[i], in_sem.at[i]
        ).wait()
        pltpu.make_async_copy(
            bufs[i], out_hbm.at[pl.ds(i * _CHUNK, _CHUNK)], out_sem.at[i]
        ).start()
    for i in range(_NCHUNK):
        pltpu.make_async_copy(
            bufs[i], out_hbm.at[pl.ds(i * _CHUNK, _CHUNK)], out_sem.at[i]
        ).wait()


def kernel(query, relation_weight):
    out, mask = pl.pallas_call(
        _copy_kernel,
        in_specs=[pl.BlockSpec(memory_space=pl.ANY)],
        out_specs=[
            pl.BlockSpec(memory_space=pl.ANY),
            pl.BlockSpec(memory_space=pltpu.MemorySpace.VMEM),
        ],
        out_shape=[
            jax.ShapeDtypeStruct((_ROWS, _DIM), jnp.float32),
            jax.ShapeDtypeStruct((_DIM, _DIM), jnp.bool_),
        ],
        scratch_shapes=(
            [pltpu.VMEM((_CHUNK, _DIM), jnp.float32) for _ in range(_NCHUNK)]
            + [
                pltpu.SemaphoreType.DMA((_NCHUNK,)),
                pltpu.SemaphoreType.DMA((_NCHUNK,)),
            ]
        ),
    )(query)
    return (out.reshape(_ROWS, 1, _DIM), mask.reshape(_ROWS, 1))


# speedup vs baseline: 1.0851x; 1.0851x over previous
import jax
import jax.numpy as jnp
from jax.experimental import pallas as pl


_ROWS = 16384
_DIM = 128
_BLOCK = 8192


def _copy_kernel(q_ref, out_ref, mask_ref):
    out_ref[...] = q_ref[...]

    @pl.when(pl.program_id(0) == 0)
    def _():
        mask_ref[...] = jnp.ones(mask_ref.shape, dtype=jnp.bool_)


def kernel(query, relation_weight):
    out, mask = pl.pallas_call(
        _copy_kernel,
        grid=(_ROWS // _BLOCK,),
        in_specs=[pl.BlockSpec((_BLOCK, _DIM), lambda i: (i, 0))],
        out_specs=[
            pl.BlockSpec((_BLOCK, _DIM), lambda i: (i, 0)),
            pl.BlockSpec((_DIM, _DIM), lambda i: (0, 0)),
        ],
        out_shape=[
            jax.ShapeDtypeStruct((_ROWS, _DIM), jnp.float32),
            jax.ShapeDtypeStruct((_DIM, _DIM), jnp.bool_),
        ],
    )(query)
    return (out.reshape(_ROWS, 1, _DIM), mask.reshape(_ROWS, 1))
